# baseline (speedup 1.0000x reference)
import jax
import jax.numpy as jnp
from jax.experimental import pallas as pl


def _mk(scores_ref, mask_ref, out_ref):
    out_ref[...] = jnp.where(mask_ref[...], scores_ref[...], jnp.float32(-jnp.inf))


def kernel(input_ids, scores, allowed_mask):
    del input_ids
    b, v = scores.shape
    vb = 25088
    nv = pl.cdiv(v, vb)
    idx = lambda vi: (0, vi)
    return pl.pallas_call(
        _mk,
        grid=(nv,),
        in_specs=[pl.BlockSpec((b, vb), idx), pl.BlockSpec((b, vb), idx)],
        out_specs=pl.BlockSpec((b, vb), idx),
        out_shape=jax.ShapeDtypeStruct((b, v), scores.dtype),
    )(scores, allowed_mask)


# speedup vs baseline: 1.4417x; 1.4417x over previous
import jax
import jax.numpy as jnp
from jax.experimental import pallas as pl


def _mk(scores_ref, mask_ref, out_ref):
    out_ref[...] = jnp.where(mask_ref[...] != 0, scores_ref[...], jnp.float32(-jnp.inf))


def kernel(input_ids, scores, allowed_mask):
    del input_ids
    b, v = scores.shape
    m8 = allowed_mask.astype(jnp.int8)
    vb = 25088
    nv = pl.cdiv(v, vb)
    idx = lambda vi: (0, vi)
    return pl.pallas_call(
        _mk,
        grid=(nv,),
        in_specs=[pl.BlockSpec((b, vb), idx), pl.BlockSpec((b, vb), idx)],
        out_specs=pl.BlockSpec((b, vb), idx),
        out_shape=jax.ShapeDtypeStruct((b, v), scores.dtype),
    )(scores, m8)
